# Initial kernel scaffold; baseline (speedup 1.0000x reference)
import jax
import jax.numpy as jnp
from jax.experimental import pallas as pl


def kernel(x, edge_index, W1_l, b1, W1_r, bn_gamma, bn_beta, bn_mean, bn_var, W2_l, b2, W2_r):
    raise NotImplementedError("write your pallas kernel here")


# speedup vs baseline: 8.1149x; 8.1149x over previous
import functools

import jax
import jax.numpy as jnp
from jax import lax
from jax.experimental import pallas as pl
from jax.experimental.pallas import tpu as pltpu
from jax.experimental.pallas import tpu_sc as plsc

N = 10000
E = 320000
D_IN = 128
D_HID = 128
D_OUT = 40
D1 = 144
D2 = 48
NC = 2
NS = 16
TILES = NC * NS
EPT = E // TILES
K = 128
NCH = EPT // K
TAIL = EPT - NCH * K
ROWS_PER_SUB = 624
ROWS_TAIL = N - NS * ROWS_PER_SUB
ROWS_TAIL_BASE = NS * ROWS_PER_SUB

_HIGH = jax.lax.Precision.HIGHEST


def _dot(a, b):
    return jax.lax.dot_general(a, b, (((1,), (0,)), ((), ())),
                               precision=_HIGH,
                               preferred_element_type=jnp.float32)


def _tc1_body(x_ref, w1l_ref, w1r_ref, t1p_ref, xr1_ref):
    x = x_ref[...]
    t1 = _dot(x, w1l_ref[...])
    t1p_ref[...] = jnp.concatenate(
        [t1, jnp.ones((N, D1 - D_HID), jnp.float32)], axis=1)
    xr1_ref[...] = _dot(x, w1r_ref[...])


TC2_BLK = 2000


def _tc2_body(agg_ref, xr1_ref, b1_ref, g_ref, be_ref, mu_ref, var_ref,
              w2l_ref, w2r_ref, t2p_ref, hr2_ref):
    s = agg_ref[0] + agg_ref[1]
    deg = s[:, D_HID]
    inv = 1.0 / jnp.maximum(deg, 1.0)
    pre = s[:, :D_HID] * inv[:, None] + xr1_ref[...] + b1_ref[...]
    h = (pre - mu_ref[...]) * jax.lax.rsqrt(var_ref[...] + 1e-5) \
        * g_ref[...] + be_ref[...]
    h = jnp.maximum(h, 0.0)
    t2p_ref[...] = _dot(h, w2l_ref[...])
    hr2 = _dot(h, w2r_ref[...])
    cols = jax.lax.broadcasted_iota(jnp.int32, (TC2_BLK, D2), 1)
    hr2_ref[...] = jnp.where(cols == D_OUT, deg[:, None], hr2)


def _tc3_body(agg2_ref, hr2_ref, b2_ref, out_ref):
    s = agg2_ref[0] + agg2_ref[1]
    hr2 = hr2_ref[...]
    deg = hr2[:, D_OUT]
    inv = 1.0 / jnp.maximum(deg, 1.0)
    z = s[:, :D_OUT] * inv[:, None] + hr2[:, :D_OUT] + b2_ref[...]
    m = jnp.max(z, axis=1, keepdims=True)
    lse = jnp.log(jnp.sum(jnp.exp(z - m), axis=1, keepdims=True)) + m
    out_ref[...] = z - lse


def _make_sc_agg(d):
    mesh = plsc.VectorSubcoreMesh(core_axis_name="c", subcore_axis_name="s",
                                  num_cores=NC, num_subcores=NS)

    @functools.partial(
        pl.kernel,
        out_type=jax.ShapeDtypeStruct((NC, N, d), jnp.float32),
        mesh=mesh,
        scratch_types=[
            pltpu.VMEM((NCH, K), jnp.int32),
            pltpu.VMEM((NCH, K), jnp.int32),
            pltpu.VMEM((TAIL,), jnp.int32),
            pltpu.VMEM((TAIL,), jnp.int32),
            pltpu.VMEM((K, d), jnp.float32),
            pltpu.VMEM((TAIL, d), jnp.float32),
            pltpu.VMEM_SHARED((N, d), jnp.float32),
            pltpu.SemaphoreType.DMA,
        ],
        compiler_params=pltpu.CompilerParams(use_tc_tiling_on_sc=False),
    )
    def sc_agg(table_hbm, srcm_hbm, dstm_hbm, srct_hbm, dstt_hbm, zeros_hbm,
               out_hbm, src_v, dst_v, srct_v, dstt_v, rows_v, rowst_v,
               acc, sem):
        cid = lax.axis_index("c")
        sid = lax.axis_index("s")
        pltpu.sync_copy(srcm_hbm.at[cid, sid], src_v)
        pltpu.sync_copy(dstm_hbm.at[cid, sid], dst_v)
        pltpu.sync_copy(srct_hbm.at[cid, sid], srct_v)
        pltpu.sync_copy(dstt_hbm.at[cid, sid], dstt_v)
        base = pl.multiple_of(sid * ROWS_PER_SUB, 8)
        pltpu.sync_copy(zeros_hbm.at[pl.ds(base, ROWS_PER_SUB)],
                        acc.at[pl.ds(base, ROWS_PER_SUB)])

        @pl.when(sid == 0)
        def _():
            pltpu.sync_copy(zeros_hbm.at[pl.ds(ROWS_TAIL_BASE, ROWS_TAIL)],
                            acc.at[pl.ds(ROWS_TAIL_BASE, ROWS_TAIL)])

        plsc.subcore_barrier()

        @pl.loop(0, NCH)
        def _(j):
            pltpu.async_copy(table_hbm.at[src_v.at[j]], rows_v, sem).wait()
            pltpu.sync_copy(rows_v, acc.at[dst_v.at[j]], add=True)

        pltpu.async_copy(table_hbm.at[srct_v], rowst_v, sem).wait()
        pltpu.sync_copy(rowst_v, acc.at[dstt_v], add=True)
        plsc.subcore_barrier()
        pltpu.sync_copy(acc.at[pl.ds(base, ROWS_PER_SUB)],
                        out_hbm.at[cid, pl.ds(base, ROWS_PER_SUB)])

        @pl.when(sid == 0)
        def _():
            pltpu.sync_copy(acc.at[pl.ds(ROWS_TAIL_BASE, ROWS_TAIL)],
                            out_hbm.at[cid, pl.ds(ROWS_TAIL_BASE, ROWS_TAIL)])

    return sc_agg


_sc_agg1 = _make_sc_agg(D1)
_sc_agg2 = _make_sc_agg(D2)


def _split_edges(idx):
    per = idx.reshape(NC, NS, EPT)
    main = per[:, :, :NCH * K].reshape(NC, NS, NCH, K)
    tail = per[:, :, NCH * K:]
    return main, tail


def kernel(x, edge_index, W1_l, b1, W1_r, bn_gamma, bn_beta, bn_mean, bn_var,
           W2_l, b2, W2_r):
    srcm, srct = _split_edges(edge_index[0])
    dstm, dstt = _split_edges(edge_index[1])
    zeros1 = jnp.zeros((N, D1), jnp.float32)
    zeros2 = jnp.zeros((N, D2), jnp.float32)
    w2l_pad = jnp.pad(W2_l, ((0, 0), (0, D2 - D_OUT)))
    w2r_pad = jnp.pad(W2_r, ((0, 0), (0, D2 - D_OUT)))

    t1p, xr1 = pl.pallas_call(
        _tc1_body,
        out_shape=[jax.ShapeDtypeStruct((N, D1), jnp.float32),
                   jax.ShapeDtypeStruct((N, D_HID), jnp.float32)],
    )(x, W1_l, W1_r)

    agg1 = _sc_agg1(t1p, srcm, dstm, srct, dstt, zeros1)

    param_spec = pl.BlockSpec((1, D_HID), lambda i: (0, 0))
    t2p, hr2 = pl.pallas_call(
        _tc2_body,
        grid=(N // TC2_BLK,),
        in_specs=[
            pl.BlockSpec((NC, TC2_BLK, D1), lambda i: (0, i, 0)),
            pl.BlockSpec((TC2_BLK, D_HID), lambda i: (i, 0)),
            param_spec, param_spec, param_spec, param_spec, param_spec,
            pl.BlockSpec((D_HID, D2), lambda i: (0, 0)),
            pl.BlockSpec((D_HID, D2), lambda i: (0, 0)),
        ],
        out_specs=[
            pl.BlockSpec((TC2_BLK, D2), lambda i: (i, 0)),
            pl.BlockSpec((TC2_BLK, D2), lambda i: (i, 0)),
        ],
        out_shape=[jax.ShapeDtypeStruct((N, D2), jnp.float32),
                   jax.ShapeDtypeStruct((N, D2), jnp.float32)],
    )(agg1, xr1, b1.reshape(1, -1), bn_gamma.reshape(1, -1),
      bn_beta.reshape(1, -1), bn_mean.reshape(1, -1), bn_var.reshape(1, -1),
      w2l_pad, w2r_pad)

    agg2 = _sc_agg2(t2p, srcm, dstm, srct, dstt, zeros2)

    out = pl.pallas_call(
        _tc3_body,
        out_shape=jax.ShapeDtypeStruct((N, D_OUT), jnp.float32),
    )(agg2, hr2, b2.reshape(1, -1))

    return out


# speedup vs baseline: 10.5432x; 1.2992x over previous
import functools

import jax
import jax.numpy as jnp
from jax import lax
from jax.experimental import pallas as pl
from jax.experimental.pallas import tpu as pltpu
from jax.experimental.pallas import tpu_sc as plsc

N = 10000
E = 320000
D_IN = 128
D_HID = 128
D_OUT = 40
D1 = 144
D2 = 48
NC = 2
NS = 16
TILES = NC * NS
EPT = E // TILES
TAIL = 16
ROWS_PER_SUB = 624
ROWS_TAIL = N - NS * ROWS_PER_SUB
ROWS_TAIL_BASE = NS * ROWS_PER_SUB

_HIGH = jax.lax.Precision.HIGHEST


def _dot(a, b):
    return jax.lax.dot_general(a, b, (((1,), (0,)), ((), ())),
                               precision=_HIGH,
                               preferred_element_type=jnp.float32)


def _tc1_body(x_ref, w1l_ref, w1r_ref, t1p_ref, xr1_ref):
    x = x_ref[...]
    t1 = _dot(x, w1l_ref[...])
    t1p_ref[...] = jnp.concatenate(
        [t1, jnp.ones((N, D1 - D_HID), jnp.float32)], axis=1)
    xr1_ref[...] = _dot(x, w1r_ref[...])


TC2_BLK = 2000


def _tc2_body(agg_ref, xr1_ref, b1_ref, g_ref, be_ref, mu_ref, var_ref,
              w2l_ref, w2r_ref, t2p_ref, hr2_ref):
    s = agg_ref[0] + agg_ref[1]
    deg = s[:, D_HID]
    inv = 1.0 / jnp.maximum(deg, 1.0)
    pre = s[:, :D_HID] * inv[:, None] + xr1_ref[...] + b1_ref[...]
    h = (pre - mu_ref[...]) * jax.lax.rsqrt(var_ref[...] + 1e-5) \
        * g_ref[...] + be_ref[...]
    h = jnp.maximum(h, 0.0)
    t2p_ref[...] = _dot(h, w2l_ref[...])
    hr2 = _dot(h, w2r_ref[...])
    cols = jax.lax.broadcasted_iota(jnp.int32, (TC2_BLK, D2), 1)
    hr2_ref[...] = jnp.where(cols == D_OUT, deg[:, None], hr2)


def _tc3_body(agg2_ref, hr2_ref, b2_ref, out_ref):
    s = agg2_ref[0] + agg2_ref[1]
    hr2 = hr2_ref[...]
    deg = hr2[:, D_OUT]
    inv = 1.0 / jnp.maximum(deg, 1.0)
    z = s[:, :D_OUT] * inv[:, None] + hr2[:, :D_OUT] + b2_ref[...]
    m = jnp.max(z, axis=1, keepdims=True)
    lse = jnp.log(jnp.sum(jnp.exp(z - m), axis=1, keepdims=True)) + m
    out_ref[...] = z - lse


def _make_sc_agg(d, k):
    ncheck = EPT - TAIL
    assert ncheck % k == 0 and (ncheck // k) % 2 == 0
    nch = ncheck // k
    mesh = plsc.VectorSubcoreMesh(core_axis_name="c", subcore_axis_name="s",
                                  num_cores=NC, num_subcores=NS)

    @functools.partial(
        pl.kernel,
        out_type=jax.ShapeDtypeStruct((NC, N, d), jnp.float32),
        mesh=mesh,
        scratch_types=[
            pltpu.VMEM((nch, k), jnp.int32),
            pltpu.VMEM((nch, k), jnp.int32),
            pltpu.VMEM((TAIL,), jnp.int32),
            pltpu.VMEM((TAIL,), jnp.int32),
            pltpu.VMEM((k, d), jnp.float32),
            pltpu.VMEM((k, d), jnp.float32),
            pltpu.VMEM((TAIL, d), jnp.float32),
            pltpu.VMEM_SHARED((N, d), jnp.float32),
            pltpu.SemaphoreType.DMA,
            pltpu.SemaphoreType.DMA,
        ],
        compiler_params=pltpu.CompilerParams(use_tc_tiling_on_sc=False),
    )
    def sc_agg(table_hbm, srcm_hbm, dstm_hbm, srct_hbm, dstt_hbm, zeros_hbm,
               out_hbm, src_v, dst_v, srct_v, dstt_v, rows0_v, rows1_v,
               rowst_v, acc, sem0, sem1):
        cid = lax.axis_index("c")
        sid = lax.axis_index("s")
        pltpu.sync_copy(srcm_hbm.at[cid, sid], src_v)
        pltpu.sync_copy(dstm_hbm.at[cid, sid], dst_v)
        pltpu.sync_copy(srct_hbm.at[cid, sid], srct_v)
        pltpu.sync_copy(dstt_hbm.at[cid, sid], dstt_v)
        rows = (rows0_v, rows1_v)
        sems = (sem0, sem1)
        for b in range(2):
            pltpu.async_copy(table_hbm.at[src_v.at[b]], rows[b], sems[b])
        base = pl.multiple_of(sid * ROWS_PER_SUB, 8)
        pltpu.sync_copy(zeros_hbm.at[pl.ds(base, ROWS_PER_SUB)],
                        acc.at[pl.ds(base, ROWS_PER_SUB)])

        @pl.when(sid == 0)
        def _():
            pltpu.sync_copy(zeros_hbm.at[pl.ds(ROWS_TAIL_BASE, ROWS_TAIL)],
                            acc.at[pl.ds(ROWS_TAIL_BASE, ROWS_TAIL)])

        plsc.subcore_barrier()

        @pl.loop(0, nch, step=2)
        def _(g):
            for b in range(2):
                j = g + b
                pltpu.make_async_copy(table_hbm.at[src_v.at[j]],
                                      rows[b], sems[b]).wait()
                pltpu.sync_copy(rows[b], acc.at[dst_v.at[j]], add=True)
                nxt = j + 2

                @pl.when(nxt < nch)
                def _():
                    pltpu.async_copy(table_hbm.at[src_v.at[nxt]],
                                     rows[b], sems[b])

        pltpu.async_copy(table_hbm.at[srct_v], rowst_v, sem0).wait()
        pltpu.sync_copy(rowst_v, acc.at[dstt_v], add=True)
        plsc.subcore_barrier()
        pltpu.sync_copy(acc.at[pl.ds(base, ROWS_PER_SUB)],
                        out_hbm.at[cid, pl.ds(base, ROWS_PER_SUB)])

        @pl.when(sid == 0)
        def _():
            pltpu.sync_copy(acc.at[pl.ds(ROWS_TAIL_BASE, ROWS_TAIL)],
                            out_hbm.at[cid, pl.ds(ROWS_TAIL_BASE, ROWS_TAIL)])

    return sc_agg


_sc_agg1 = _make_sc_agg(D1, 64)
_sc_agg2 = _make_sc_agg(D2, 128)


def _split_edges(idx, k):
    per = idx.reshape(NC, NS, EPT)
    main = per[:, :, :EPT - TAIL].reshape(NC, NS, (EPT - TAIL) // k, k)
    tail = per[:, :, EPT - TAIL:]
    return main, tail


def kernel(x, edge_index, W1_l, b1, W1_r, bn_gamma, bn_beta, bn_mean, bn_var,
           W2_l, b2, W2_r):
    srcm1, srct = _split_edges(edge_index[0], 64)
    dstm1, dstt = _split_edges(edge_index[1], 64)
    srcm2 = srcm1.reshape(NC, NS, -1, 128)
    dstm2 = dstm1.reshape(NC, NS, -1, 128)
    zeros1 = jnp.zeros((N, D1), jnp.float32)
    zeros2 = jnp.zeros((N, D2), jnp.float32)
    w2l_pad = jnp.pad(W2_l, ((0, 0), (0, D2 - D_OUT)))
    w2r_pad = jnp.pad(W2_r, ((0, 0), (0, D2 - D_OUT)))

    t1p, xr1 = pl.pallas_call(
        _tc1_body,
        out_shape=[jax.ShapeDtypeStruct((N, D1), jnp.float32),
                   jax.ShapeDtypeStruct((N, D_HID), jnp.float32)],
    )(x, W1_l, W1_r)

    agg1 = _sc_agg1(t1p, srcm1, dstm1, srct, dstt, zeros1)

    param_spec = pl.BlockSpec((1, D_HID), lambda i: (0, 0))
    t2p, hr2 = pl.pallas_call(
        _tc2_body,
        grid=(N // TC2_BLK,),
        in_specs=[
            pl.BlockSpec((NC, TC2_BLK, D1), lambda i: (0, i, 0)),
            pl.BlockSpec((TC2_BLK, D_HID), lambda i: (i, 0)),
            param_spec, param_spec, param_spec, param_spec, param_spec,
            pl.BlockSpec((D_HID, D2), lambda i: (0, 0)),
            pl.BlockSpec((D_HID, D2), lambda i: (0, 0)),
        ],
        out_specs=[
            pl.BlockSpec((TC2_BLK, D2), lambda i: (i, 0)),
            pl.BlockSpec((TC2_BLK, D2), lambda i: (i, 0)),
        ],
        out_shape=[jax.ShapeDtypeStruct((N, D2), jnp.float32),
                   jax.ShapeDtypeStruct((N, D2), jnp.float32)],
    )(agg1, xr1, b1.reshape(1, -1), bn_gamma.reshape(1, -1),
      bn_beta.reshape(1, -1), bn_mean.reshape(1, -1), bn_var.reshape(1, -1),
      w2l_pad, w2r_pad)

    agg2 = _sc_agg2(t2p, srcm2, dstm2, srct, dstt, zeros2)

    out = pl.pallas_call(
        _tc3_body,
        out_shape=jax.ShapeDtypeStruct((N, D_OUT), jnp.float32),
    )(agg2, hr2, b2.reshape(1, -1))

    return out


# speedup vs baseline: 11.1533x; 1.0579x over previous
import functools

import jax
import jax.numpy as jnp
from jax import lax
from jax.experimental import pallas as pl
from jax.experimental.pallas import tpu as pltpu
from jax.experimental.pallas import tpu_sc as plsc

N = 10000
E = 320000
D_IN = 128
D_HID = 128
D_OUT = 40
D1 = 144
D2 = 48
NC = 2
NS = 16
TILES = NC * NS
EPT = E // TILES
TAIL = 16
ROWS_PER_SUB = 624
ROWS_TAIL = N - NS * ROWS_PER_SUB
ROWS_TAIL_BASE = NS * ROWS_PER_SUB

_HIGH = jax.lax.Precision.HIGHEST


def _dot(a, b):
    return jax.lax.dot_general(a, b, (((1,), (0,)), ((), ())),
                               precision=_HIGH,
                               preferred_element_type=jnp.float32)


TC1_BLK = 2000


def _tc1_body(x_ref, w1l_ref, w1r_ref, t1p_ref, xr1_ref):
    x = x_ref[...]
    t1 = _dot(x, w1l_ref[...])
    t1p_ref[...] = jnp.concatenate(
        [t1, jnp.ones((TC1_BLK, D1 - D_HID), jnp.float32)], axis=1)
    xr1_ref[...] = _dot(x, w1r_ref[...])


TC2_BLK = 2000


def _tc2_body(agg_ref, xr1_ref, b1_ref, g_ref, be_ref, mu_ref, var_ref,
              w2l_ref, w2r_ref, t2p_ref, hr2_ref):
    s = agg_ref[0] + agg_ref[1]
    deg = s[:, D_HID]
    inv = 1.0 / jnp.maximum(deg, 1.0)
    pre = s[:, :D_HID] * inv[:, None] + xr1_ref[...] + b1_ref[...]
    h = (pre - mu_ref[...]) * jax.lax.rsqrt(var_ref[...] + 1e-5) \
        * g_ref[...] + be_ref[...]
    h = jnp.maximum(h, 0.0)
    t2p_ref[...] = _dot(h, w2l_ref[...])
    hr2 = _dot(h, w2r_ref[...])
    cols = jax.lax.broadcasted_iota(jnp.int32, (TC2_BLK, D2), 1)
    hr2_ref[...] = jnp.where(cols == D_OUT, deg[:, None], hr2)


TC3_BLK = 2000


def _tc3_body(agg2_ref, hr2_ref, b2_ref, out_ref):
    s = agg2_ref[0] + agg2_ref[1]
    hr2 = hr2_ref[...]
    deg = hr2[:, D_OUT]
    inv = 1.0 / jnp.maximum(deg, 1.0)
    z = s[:, :D_OUT] * inv[:, None] + hr2[:, :D_OUT] + b2_ref[...]
    m = jnp.max(z, axis=1, keepdims=True)
    lse = jnp.log(jnp.sum(jnp.exp(z - m), axis=1, keepdims=True)) + m
    out_ref[...] = z - lse


def _make_sc_agg(d, k, nbuf):
    ncheck = EPT - TAIL
    assert ncheck % k == 0 and (ncheck // k) % nbuf == 0
    nch = ncheck // k
    mesh = plsc.VectorSubcoreMesh(core_axis_name="c", subcore_axis_name="s",
                                  num_cores=NC, num_subcores=NS)

    @functools.partial(
        pl.kernel,
        out_type=jax.ShapeDtypeStruct((NC, N, d), jnp.float32),
        mesh=mesh,
        scratch_types=[
            pltpu.VMEM((nch, k), jnp.int32),
            pltpu.VMEM((nch, k), jnp.int32),
            pltpu.VMEM((TAIL,), jnp.int32),
            pltpu.VMEM((TAIL,), jnp.int32),
            [pltpu.VMEM((k, d), jnp.float32) for _ in range(nbuf)],
            pltpu.VMEM((TAIL, d), jnp.float32),
            pltpu.VMEM_SHARED((N, d), jnp.float32),
            [pltpu.SemaphoreType.DMA for _ in range(nbuf)],
        ],
        compiler_params=pltpu.CompilerParams(use_tc_tiling_on_sc=False),
    )
    def sc_agg(table_hbm, srcm_hbm, dstm_hbm, srct_hbm, dstt_hbm, zeros_hbm,
               out_hbm, src_v, dst_v, srct_v, dstt_v, rows, rowst_v,
               acc, sems):
        cid = lax.axis_index("c")
        sid = lax.axis_index("s")
        pltpu.sync_copy(srcm_hbm.at[cid, sid], src_v)
        pltpu.sync_copy(dstm_hbm.at[cid, sid], dst_v)
        pltpu.sync_copy(srct_hbm.at[cid, sid], srct_v)
        pltpu.sync_copy(dstt_hbm.at[cid, sid], dstt_v)
        for b in range(nbuf):
            pltpu.async_copy(table_hbm.at[src_v.at[b]], rows[b], sems[b])
        base = pl.multiple_of(sid * ROWS_PER_SUB, 8)
        pltpu.sync_copy(zeros_hbm.at[pl.ds(base, ROWS_PER_SUB)],
                        acc.at[pl.ds(base, ROWS_PER_SUB)])

        @pl.when(sid == 0)
        def _():
            pltpu.sync_copy(zeros_hbm.at[pl.ds(ROWS_TAIL_BASE, ROWS_TAIL)],
                            acc.at[pl.ds(ROWS_TAIL_BASE, ROWS_TAIL)])

        plsc.subcore_barrier()

        @pl.loop(0, nch, step=nbuf)
        def _(g):
            for b in range(nbuf):
                j = g + b
                pltpu.make_async_copy(table_hbm.at[src_v.at[j]],
                                      rows[b], sems[b]).wait()
                pltpu.sync_copy(rows[b], acc.at[dst_v.at[j]], add=True)
                nxt = j + nbuf

                @pl.when(nxt < nch)
                def _():
                    pltpu.async_copy(table_hbm.at[src_v.at[nxt]],
                                     rows[b], sems[b])

        pltpu.async_copy(table_hbm.at[srct_v], rowst_v, sems[0]).wait()
        pltpu.sync_copy(rowst_v, acc.at[dstt_v], add=True)
        plsc.subcore_barrier()
        pltpu.sync_copy(acc.at[pl.ds(base, ROWS_PER_SUB)],
                        out_hbm.at[cid, pl.ds(base, ROWS_PER_SUB)])

        @pl.when(sid == 0)
        def _():
            pltpu.sync_copy(acc.at[pl.ds(ROWS_TAIL_BASE, ROWS_TAIL)],
                            out_hbm.at[cid, pl.ds(ROWS_TAIL_BASE, ROWS_TAIL)])

    return sc_agg


_sc_agg1 = _make_sc_agg(D1, 64, 2)
_sc_agg2 = _make_sc_agg(D2, 128, 3)


def _split_edges(idx, k):
    per = idx.reshape(NC, NS, EPT)
    main = per[:, :, :EPT - TAIL].reshape(NC, NS, (EPT - TAIL) // k, k)
    tail = per[:, :, EPT - TAIL:]
    return main, tail


def kernel(x, edge_index, W1_l, b1, W1_r, bn_gamma, bn_beta, bn_mean, bn_var,
           W2_l, b2, W2_r):
    srcm1, srct = _split_edges(edge_index[0], 64)
    dstm1, dstt = _split_edges(edge_index[1], 64)
    srcm2 = srcm1.reshape(NC, NS, -1, 128)
    dstm2 = dstm1.reshape(NC, NS, -1, 128)
    zeros1 = jnp.zeros((N, D1), jnp.float32)
    zeros2 = jnp.zeros((N, D2), jnp.float32)
    w2l_pad = jnp.pad(W2_l, ((0, 0), (0, D2 - D_OUT)))
    w2r_pad = jnp.pad(W2_r, ((0, 0), (0, D2 - D_OUT)))

    t1p, xr1 = pl.pallas_call(
        _tc1_body,
        grid=(N // TC1_BLK,),
        in_specs=[
            pl.BlockSpec((TC1_BLK, D_IN), lambda i: (i, 0)),
            pl.BlockSpec((D_IN, D_HID), lambda i: (0, 0)),
            pl.BlockSpec((D_IN, D_HID), lambda i: (0, 0)),
        ],
        out_specs=[
            pl.BlockSpec((TC1_BLK, D1), lambda i: (i, 0)),
            pl.BlockSpec((TC1_BLK, D_HID), lambda i: (i, 0)),
        ],
        out_shape=[jax.ShapeDtypeStruct((N, D1), jnp.float32),
                   jax.ShapeDtypeStruct((N, D_HID), jnp.float32)],
    )(x, W1_l, W1_r)

    agg1 = _sc_agg1(t1p, srcm1, dstm1, srct, dstt, zeros1)

    param_spec = pl.BlockSpec((1, D_HID), lambda i: (0, 0))
    t2p, hr2 = pl.pallas_call(
        _tc2_body,
        grid=(N // TC2_BLK,),
        in_specs=[
            pl.BlockSpec((NC, TC2_BLK, D1), lambda i: (0, i, 0)),
            pl.BlockSpec((TC2_BLK, D_HID), lambda i: (i, 0)),
            param_spec, param_spec, param_spec, param_spec, param_spec,
            pl.BlockSpec((D_HID, D2), lambda i: (0, 0)),
            pl.BlockSpec((D_HID, D2), lambda i: (0, 0)),
        ],
        out_specs=[
            pl.BlockSpec((TC2_BLK, D2), lambda i: (i, 0)),
            pl.BlockSpec((TC2_BLK, D2), lambda i: (i, 0)),
        ],
        out_shape=[jax.ShapeDtypeStruct((N, D2), jnp.float32),
                   jax.ShapeDtypeStruct((N, D2), jnp.float32)],
    )(agg1, xr1, b1.reshape(1, -1), bn_gamma.reshape(1, -1),
      bn_beta.reshape(1, -1), bn_mean.reshape(1, -1), bn_var.reshape(1, -1),
      w2l_pad, w2r_pad)

    agg2 = _sc_agg2(t2p, srcm2, dstm2, srct, dstt, zeros2)

    out = pl.pallas_call(
        _tc3_body,
        grid=(N // TC3_BLK,),
        in_specs=[
            pl.BlockSpec((NC, TC3_BLK, D2), lambda i: (0, i, 0)),
            pl.BlockSpec((TC3_BLK, D2), lambda i: (i, 0)),
            pl.BlockSpec((1, D_OUT), lambda i: (0, 0)),
        ],
        out_specs=pl.BlockSpec((TC3_BLK, D_OUT), lambda i: (i, 0)),
        out_shape=jax.ShapeDtypeStruct((N, D_OUT), jnp.float32),
    )(agg2, hr2, b2.reshape(1, -1))

    return out


# speedup vs baseline: 12.1190x; 1.0866x over previous
import functools

import jax
import jax.numpy as jnp
from jax import lax
from jax.experimental import pallas as pl
from jax.experimental.pallas import tpu as pltpu
from jax.experimental.pallas import tpu_sc as plsc

N = 10000
E = 320000
D_IN = 128
D_HID = 128
D_OUT = 40
D1 = 144
D2 = 48
NC = 2
NS = 16
TILES = NC * NS
EPT = E // TILES
TAIL = 16
ROWS_PER_SUB = 624
ROWS_TAIL = N - NS * ROWS_PER_SUB
ROWS_TAIL_BASE = NS * ROWS_PER_SUB

_HIGH = jax.lax.Precision.HIGHEST


def _dot(a, b):
    return jax.lax.dot_general(a, b, (((1,), (0,)), ((), ())),
                               precision=_HIGH,
                               preferred_element_type=jnp.float32)


TC1_BLK = 2000


def _tc1_body(x_ref, w1l_ref, w1r_ref, t1p_ref, xr1_ref):
    x = x_ref[...]
    t1 = _dot(x, w1l_ref[...])
    t1p_ref[...] = jnp.concatenate(
        [t1, jnp.ones((TC1_BLK, D1 - D_HID), jnp.float32)], axis=1)
    xr1_ref[...] = _dot(x, w1r_ref[...])


TC2_BLK = 2000


def _tc2_body(agg_ref, xr1_ref, b1_ref, g_ref, be_ref, mu_ref, var_ref,
              w2l_ref, w2r_ref, t2p_ref, hr2_ref):
    s = agg_ref[0] + agg_ref[1]
    deg = s[:, D_HID]
    inv = 1.0 / jnp.maximum(deg, 1.0)
    pre = s[:, :D_HID] * inv[:, None] + xr1_ref[...] + b1_ref[...]
    h = (pre - mu_ref[...]) * jax.lax.rsqrt(var_ref[...] + 1e-5) \
        * g_ref[...] + be_ref[...]
    h = jnp.maximum(h, 0.0)
    t2p_ref[...] = _dot(h, w2l_ref[...])
    hr2 = _dot(h, w2r_ref[...])
    cols = jax.lax.broadcasted_iota(jnp.int32, (TC2_BLK, D2), 1)
    hr2_ref[...] = jnp.where(cols == D_OUT, deg[:, None], hr2)


TC3_BLK = 2000


def _tc3_body(agg2_ref, hr2_ref, b2_ref, out_ref):
    s = agg2_ref[0] + agg2_ref[1]
    hr2 = hr2_ref[...]
    deg = hr2[:, D_OUT]
    inv = 1.0 / jnp.maximum(deg, 1.0)
    z = s[:, :D_OUT] * inv[:, None] + hr2[:, :D_OUT] + b2_ref[...]
    m = jnp.max(z, axis=1, keepdims=True)
    lse = jnp.log(jnp.sum(jnp.exp(z - m), axis=1, keepdims=True)) + m
    out_ref[...] = z - lse


def _make_sc_agg(d, k, nbuf):
    ncheck = EPT - TAIL
    assert ncheck % k == 0 and (ncheck // k) % nbuf == 0
    nch = ncheck // k
    mesh = plsc.VectorSubcoreMesh(core_axis_name="c", subcore_axis_name="s",
                                  num_cores=NC, num_subcores=NS)

    @functools.partial(
        pl.kernel,
        out_type=jax.ShapeDtypeStruct((NC, N, d), jnp.float32),
        mesh=mesh,
        scratch_types=[
            pltpu.VMEM((nch, k), jnp.int32),
            pltpu.VMEM((nch, k), jnp.int32),
            pltpu.VMEM((TAIL,), jnp.int32),
            pltpu.VMEM((TAIL,), jnp.int32),
            [pltpu.VMEM((k, d), jnp.float32) for _ in range(nbuf)],
            pltpu.VMEM((TAIL, d), jnp.float32),
            pltpu.VMEM_SHARED((N, d), jnp.float32),
            [pltpu.SemaphoreType.DMA for _ in range(nbuf)],
            [pltpu.SemaphoreType.DMA for _ in range(nbuf)],
        ],
        compiler_params=pltpu.CompilerParams(use_tc_tiling_on_sc=False),
    )
    def sc_agg(table_hbm, srcm_hbm, dstm_hbm, srct_hbm, dstt_hbm, zeros_hbm,
               out_hbm, src_v, dst_v, srct_v, dstt_v, rows, rowst_v,
               acc, gsems, ssems):
        cid = lax.axis_index("c")
        sid = lax.axis_index("s")
        pltpu.sync_copy(srcm_hbm.at[cid, sid], src_v)
        pltpu.sync_copy(dstm_hbm.at[cid, sid], dst_v)
        pltpu.sync_copy(srct_hbm.at[cid, sid], srct_v)
        pltpu.sync_copy(dstt_hbm.at[cid, sid], dstt_v)
        for b in range(nbuf):
            pltpu.async_copy(table_hbm.at[src_v.at[b]], rows[b], gsems[b])
        base = pl.multiple_of(sid * ROWS_PER_SUB, 8)
        pltpu.sync_copy(zeros_hbm.at[pl.ds(base, ROWS_PER_SUB)],
                        acc.at[pl.ds(base, ROWS_PER_SUB)])

        @pl.when(sid == 0)
        def _():
            pltpu.sync_copy(zeros_hbm.at[pl.ds(ROWS_TAIL_BASE, ROWS_TAIL)],
                            acc.at[pl.ds(ROWS_TAIL_BASE, ROWS_TAIL)])

        plsc.subcore_barrier()

        @pl.loop(0, nch, step=nbuf)
        def _(g):
            for b in range(nbuf):
                j = g + b
                pltpu.make_async_copy(table_hbm.at[src_v.at[j]],
                                      rows[b], gsems[b]).wait()
                pltpu.async_copy(rows[b], acc.at[dst_v.at[j]], ssems[b],
                                 add=True)
                nxt = j + nbuf

                @pl.when(nxt < nch)
                def _():
                    pltpu.make_async_copy(rows[b], acc.at[dst_v.at[j]],
                                          ssems[b]).wait()
                    pltpu.async_copy(table_hbm.at[src_v.at[nxt]],
                                     rows[b], gsems[b])

        for b in range(nbuf):
            pltpu.make_async_copy(rows[b], acc.at[dst_v.at[nch - nbuf + b]],
                                  ssems[b]).wait()
        pltpu.async_copy(table_hbm.at[srct_v], rowst_v, gsems[0]).wait()
        pltpu.sync_copy(rowst_v, acc.at[dstt_v], add=True)
        plsc.subcore_barrier()
        pltpu.sync_copy(acc.at[pl.ds(base, ROWS_PER_SUB)],
                        out_hbm.at[cid, pl.ds(base, ROWS_PER_SUB)])

        @pl.when(sid == 0)
        def _():
            pltpu.sync_copy(acc.at[pl.ds(ROWS_TAIL_BASE, ROWS_TAIL)],
                            out_hbm.at[cid, pl.ds(ROWS_TAIL_BASE, ROWS_TAIL)])

    return sc_agg


_sc_agg1 = _make_sc_agg(D1, 32, 4)
_sc_agg2 = _make_sc_agg(D2, 96, 4)


def _split_edges(idx, k):
    per = idx.reshape(NC, NS, EPT)
    main = per[:, :, :EPT - TAIL].reshape(NC, NS, (EPT - TAIL) // k, k)
    tail = per[:, :, EPT - TAIL:]
    return main, tail


def kernel(x, edge_index, W1_l, b1, W1_r, bn_gamma, bn_beta, bn_mean, bn_var,
           W2_l, b2, W2_r):
    srcm1, srct = _split_edges(edge_index[0], 32)
    dstm1, dstt = _split_edges(edge_index[1], 32)
    srcm2 = srcm1.reshape(NC, NS, -1, 96)
    dstm2 = dstm1.reshape(NC, NS, -1, 96)
    zeros1 = jnp.zeros((N, D1), jnp.float32)
    zeros2 = jnp.zeros((N, D2), jnp.float32)
    w2l_pad = jnp.pad(W2_l, ((0, 0), (0, D2 - D_OUT)))
    w2r_pad = jnp.pad(W2_r, ((0, 0), (0, D2 - D_OUT)))

    t1p, xr1 = pl.pallas_call(
        _tc1_body,
        grid=(N // TC1_BLK,),
        in_specs=[
            pl.BlockSpec((TC1_BLK, D_IN), lambda i: (i, 0)),
            pl.BlockSpec((D_IN, D_HID), lambda i: (0, 0)),
            pl.BlockSpec((D_IN, D_HID), lambda i: (0, 0)),
        ],
        out_specs=[
            pl.BlockSpec((TC1_BLK, D1), lambda i: (i, 0)),
            pl.BlockSpec((TC1_BLK, D_HID), lambda i: (i, 0)),
        ],
        out_shape=[jax.ShapeDtypeStruct((N, D1), jnp.float32),
                   jax.ShapeDtypeStruct((N, D_HID), jnp.float32)],
    )(x, W1_l, W1_r)

    agg1 = _sc_agg1(t1p, srcm1, dstm1, srct, dstt, zeros1)

    param_spec = pl.BlockSpec((1, D_HID), lambda i: (0, 0))
    t2p, hr2 = pl.pallas_call(
        _tc2_body,
        grid=(N // TC2_BLK,),
        in_specs=[
            pl.BlockSpec((NC, TC2_BLK, D1), lambda i: (0, i, 0)),
            pl.BlockSpec((TC2_BLK, D_HID), lambda i: (i, 0)),
            param_spec, param_spec, param_spec, param_spec, param_spec,
            pl.BlockSpec((D_HID, D2), lambda i: (0, 0)),
            pl.BlockSpec((D_HID, D2), lambda i: (0, 0)),
        ],
        out_specs=[
            pl.BlockSpec((TC2_BLK, D2), lambda i: (i, 0)),
            pl.BlockSpec((TC2_BLK, D2), lambda i: (i, 0)),
        ],
        out_shape=[jax.ShapeDtypeStruct((N, D2), jnp.float32),
                   jax.ShapeDtypeStruct((N, D2), jnp.float32)],
    )(agg1, xr1, b1.reshape(1, -1), bn_gamma.reshape(1, -1),
      bn_beta.reshape(1, -1), bn_mean.reshape(1, -1), bn_var.reshape(1, -1),
      w2l_pad, w2r_pad)

    agg2 = _sc_agg2(t2p, srcm2, dstm2, srct, dstt, zeros2)

    out = pl.pallas_call(
        _tc3_body,
        grid=(N // TC3_BLK,),
        in_specs=[
            pl.BlockSpec((NC, TC3_BLK, D2), lambda i: (0, i, 0)),
            pl.BlockSpec((TC3_BLK, D2), lambda i: (i, 0)),
            pl.BlockSpec((1, D_OUT), lambda i: (0, 0)),
        ],
        out_specs=pl.BlockSpec((TC3_BLK, D_OUT), lambda i: (i, 0)),
        out_shape=jax.ShapeDtypeStruct((N, D_OUT), jnp.float32),
    )(agg2, hr2, b2.reshape(1, -1))

    return out


# speedup vs baseline: 12.5978x; 1.0395x over previous
import functools

import jax
import jax.numpy as jnp
from jax import lax
from jax.experimental import pallas as pl
from jax.experimental.pallas import tpu as pltpu
from jax.experimental.pallas import tpu_sc as plsc

N = 10000
E = 320000
D_IN = 128
D_HID = 128
D_OUT = 40
D1 = 144
D2 = 48
NC = 2
NS = 16
TILES = NC * NS
EPT = E // TILES
ROWS_PER_SUB = 624
ROWS_TAIL = N - NS * ROWS_PER_SUB
ROWS_TAIL_BASE = NS * ROWS_PER_SUB

_HIGH = jax.lax.Precision.HIGHEST


def _dot(a, b):
    return jax.lax.dot_general(a, b, (((1,), (0,)), ((), ())),
                               precision=_HIGH,
                               preferred_element_type=jnp.float32)


TC1_BLK = 2000


def _tc1_body(x_ref, w1l_ref, w1r_ref, t1p_ref, xr1_ref):
    x = x_ref[...]
    t1 = _dot(x, w1l_ref[...])
    t1p_ref[...] = jnp.concatenate(
        [t1, jnp.ones((TC1_BLK, D1 - D_HID), jnp.float32)], axis=1)
    xr1_ref[...] = _dot(x, w1r_ref[...])


TC2_BLK = 2000


def _tc2_body(agg_ref, xr1_ref, b1_ref, g_ref, be_ref, mu_ref, var_ref,
              w2l_ref, w2r_ref, t2p_ref, hr2_ref):
    s = agg_ref[0] + agg_ref[1]
    deg = s[:, D_HID]
    inv = 1.0 / jnp.maximum(deg, 1.0)
    pre = s[:, :D_HID] * inv[:, None] + xr1_ref[...] + b1_ref[...]
    h = (pre - mu_ref[...]) * jax.lax.rsqrt(var_ref[...] + 1e-5) \
        * g_ref[...] + be_ref[...]
    h = jnp.maximum(h, 0.0)
    t2p_ref[...] = _dot(h, w2l_ref[...])
    hr2 = _dot(h, w2r_ref[...])
    cols = jax.lax.broadcasted_iota(jnp.int32, (TC2_BLK, D2), 1)
    hr2_ref[...] = jnp.where(cols == D_OUT, deg[:, None], hr2)


TC3_BLK = 2000


def _tc3_body(agg2_ref, hr2_ref, b2_ref, out_ref):
    s = agg2_ref[0] + agg2_ref[1]
    hr2 = hr2_ref[...]
    deg = hr2[:, D_OUT]
    inv = 1.0 / jnp.maximum(deg, 1.0)
    z = s[:, :D_OUT] * inv[:, None] + hr2[:, :D_OUT] + b2_ref[...]
    m = jnp.max(z, axis=1, keepdims=True)
    lse = jnp.log(jnp.sum(jnp.exp(z - m), axis=1, keepdims=True)) + m
    out_ref[...] = z - lse


def _make_sc_agg(d, k, nbuf):
    assert EPT % k == 0
    nch = EPT // k
    nfull = (nch // nbuf) * nbuf
    rem = nch - nfull
    mesh = plsc.VectorSubcoreMesh(core_axis_name="c", subcore_axis_name="s",
                                  num_cores=NC, num_subcores=NS)

    @functools.partial(
        pl.kernel,
        out_type=jax.ShapeDtypeStruct((NC, N, d), jnp.float32),
        mesh=mesh,
        scratch_types=[
            pltpu.VMEM((nch, k), jnp.int32),
            pltpu.VMEM((nch, k), jnp.int32),
            [pltpu.VMEM((k, d), jnp.float32) for _ in range(nbuf)],
            pltpu.VMEM_SHARED((N, d), jnp.float32),
            [pltpu.SemaphoreType.DMA for _ in range(nbuf)],
            [pltpu.SemaphoreType.DMA for _ in range(nbuf)],
        ],
        compiler_params=pltpu.CompilerParams(use_tc_tiling_on_sc=False),
    )
    def sc_agg(table_hbm, ei_hbm, zeros_hbm,
               out_hbm, src_v, dst_v, rows, acc, gsems, ssems):
        cid = lax.axis_index("c")
        sid = lax.axis_index("s")
        pltpu.sync_copy(ei_hbm.at[0, cid, sid], src_v)
        pltpu.sync_copy(ei_hbm.at[1, cid, sid], dst_v)
        for b in range(nbuf):
            pltpu.async_copy(table_hbm.at[src_v.at[b]], rows[b], gsems[b])
        base = pl.multiple_of(sid * ROWS_PER_SUB, 8)
        pltpu.sync_copy(zeros_hbm.at[pl.ds(base, ROWS_PER_SUB)],
                        acc.at[pl.ds(base, ROWS_PER_SUB)])

        @pl.when(sid == 0)
        def _():
            pltpu.sync_copy(zeros_hbm.at[pl.ds(ROWS_TAIL_BASE, ROWS_TAIL)],
                            acc.at[pl.ds(ROWS_TAIL_BASE, ROWS_TAIL)])

        plsc.subcore_barrier()

        def step(j, b):
            pltpu.make_async_copy(table_hbm.at[src_v.at[j]],
                                  rows[b], gsems[b]).wait()
            pltpu.async_copy(rows[b], acc.at[dst_v.at[j]], ssems[b],
                             add=True)
            nxt = j + nbuf

            @pl.when(nxt < nch)
            def _():
                pltpu.make_async_copy(rows[b], acc.at[dst_v.at[j]],
                                      ssems[b]).wait()
                pltpu.async_copy(table_hbm.at[src_v.at[nxt]],
                                 rows[b], gsems[b])

        @pl.loop(0, nfull, step=nbuf)
        def _(g):
            for b in range(nbuf):
                step(g + b, b)

        for b in range(rem):
            step(nfull + b, b)

        for b in range(nbuf):
            pltpu.make_async_copy(rows[b], acc.at[dst_v.at[nch - nbuf + b]],
                                  ssems[b]).wait()
        plsc.subcore_barrier()
        pltpu.sync_copy(acc.at[pl.ds(base, ROWS_PER_SUB)],
                        out_hbm.at[cid, pl.ds(base, ROWS_PER_SUB)])

        @pl.when(sid == 0)
        def _():
            pltpu.sync_copy(acc.at[pl.ds(ROWS_TAIL_BASE, ROWS_TAIL)],
                            out_hbm.at[cid, pl.ds(ROWS_TAIL_BASE, ROWS_TAIL)])

    return sc_agg


_sc_agg1 = _make_sc_agg(D1, 40, 3)
_sc_agg2 = _make_sc_agg(D2, 80, 5)


def kernel(x, edge_index, W1_l, b1, W1_r, bn_gamma, bn_beta, bn_mean, bn_var,
           W2_l, b2, W2_r):
    ei1 = edge_index.reshape(2, NC, NS, EPT // 40, 40)
    ei2 = edge_index.reshape(2, NC, NS, EPT // 80, 80)
    zeros1 = jnp.zeros((N, D1), jnp.float32)
    zeros2 = jnp.zeros((N, D2), jnp.float32)
    w2l_pad = jnp.pad(W2_l, ((0, 0), (0, D2 - D_OUT)))
    w2r_pad = jnp.pad(W2_r, ((0, 0), (0, D2 - D_OUT)))

    t1p, xr1 = pl.pallas_call(
        _tc1_body,
        grid=(N // TC1_BLK,),
        in_specs=[
            pl.BlockSpec((TC1_BLK, D_IN), lambda i: (i, 0)),
            pl.BlockSpec((D_IN, D_HID), lambda i: (0, 0)),
            pl.BlockSpec((D_IN, D_HID), lambda i: (0, 0)),
        ],
        out_specs=[
            pl.BlockSpec((TC1_BLK, D1), lambda i: (i, 0)),
            pl.BlockSpec((TC1_BLK, D_HID), lambda i: (i, 0)),
        ],
        out_shape=[jax.ShapeDtypeStruct((N, D1), jnp.float32),
                   jax.ShapeDtypeStruct((N, D_HID), jnp.float32)],
    )(x, W1_l, W1_r)

    agg1 = _sc_agg1(t1p, ei1, zeros1)

    param_spec = pl.BlockSpec((1, D_HID), lambda i: (0, 0))
    t2p, hr2 = pl.pallas_call(
        _tc2_body,
        grid=(N // TC2_BLK,),
        in_specs=[
            pl.BlockSpec((NC, TC2_BLK, D1), lambda i: (0, i, 0)),
            pl.BlockSpec((TC2_BLK, D_HID), lambda i: (i, 0)),
            param_spec, param_spec, param_spec, param_spec, param_spec,
            pl.BlockSpec((D_HID, D2), lambda i: (0, 0)),
            pl.BlockSpec((D_HID, D2), lambda i: (0, 0)),
        ],
        out_specs=[
            pl.BlockSpec((TC2_BLK, D2), lambda i: (i, 0)),
            pl.BlockSpec((TC2_BLK, D2), lambda i: (i, 0)),
        ],
        out_shape=[jax.ShapeDtypeStruct((N, D2), jnp.float32),
                   jax.ShapeDtypeStruct((N, D2), jnp.float32)],
    )(agg1, xr1, b1.reshape(1, -1), bn_gamma.reshape(1, -1),
      bn_beta.reshape(1, -1), bn_mean.reshape(1, -1), bn_var.reshape(1, -1),
      w2l_pad, w2r_pad)

    agg2 = _sc_agg2(t2p, ei2, zeros2)

    out = pl.pallas_call(
        _tc3_body,
        grid=(N // TC3_BLK,),
        in_specs=[
            pl.BlockSpec((NC, TC3_BLK, D2), lambda i: (0, i, 0)),
            pl.BlockSpec((TC3_BLK, D2), lambda i: (i, 0)),
            pl.BlockSpec((1, D_OUT), lambda i: (0, 0)),
        ],
        out_specs=pl.BlockSpec((TC3_BLK, D_OUT), lambda i: (i, 0)),
        out_shape=jax.ShapeDtypeStruct((N, D_OUT), jnp.float32),
    )(agg2, hr2, b2.reshape(1, -1))

    return out


# speedup vs baseline: 13.3264x; 1.0578x over previous
import functools

import jax
import jax.numpy as jnp
from jax import lax
from jax.experimental import pallas as pl
from jax.experimental.pallas import tpu as pltpu
from jax.experimental.pallas import tpu_sc as plsc

N = 10000
E = 320000
D_IN = 128
D_HID = 128
D_OUT = 40
DD = 16
D2 = 48
NC = 2
NS = 16
TILES = NC * NS
EPT = E // TILES
ROWS_PER_SUB = 624
ROWS_TAIL = N - NS * ROWS_PER_SUB
ROWS_TAIL_BASE = NS * ROWS_PER_SUB

_HIGH = jax.lax.Precision.HIGHEST


def _dot(a, b):
    return jax.lax.dot_general(a, b, (((1,), (0,)), ((), ())),
                               precision=_HIGH,
                               preferred_element_type=jnp.float32)


TC2_BLK = 2000


def _tc2_body(aggx_ref, degp_ref, x_ref, b1_ref, g_ref, be_ref, mu_ref,
              var_ref, w1l_ref, w1r_ref, w2l_ref, w2r_ref, t2p_ref, hr2_ref):
    axs = aggx_ref[0] + aggx_ref[1]
    deg = degp_ref[0, :, 0] + degp_ref[1, :, 0]
    inv = 1.0 / jnp.maximum(deg, 1.0)
    x = x_ref[...]
    pre = _dot(axs * inv[:, None], w1l_ref[...]) + _dot(x, w1r_ref[...]) \
        + b1_ref[...]
    h = (pre - mu_ref[...]) * jax.lax.rsqrt(var_ref[...] + 1e-5) \
        * g_ref[...] + be_ref[...]
    h = jnp.maximum(h, 0.0)
    t2p_ref[...] = _dot(h, w2l_ref[...])
    hr2 = _dot(h, w2r_ref[...])
    cols = jax.lax.broadcasted_iota(jnp.int32, (TC2_BLK, D2), 1)
    hr2_ref[...] = jnp.where(cols == D_OUT, deg[:, None], hr2)


TC3_BLK = 2000


def _tc3_body(agg2_ref, hr2_ref, b2_ref, out_ref):
    s = agg2_ref[0] + agg2_ref[1]
    hr2 = hr2_ref[...]
    deg = hr2[:, D_OUT]
    inv = 1.0 / jnp.maximum(deg, 1.0)
    z = s[:, :D_OUT] * inv[:, None] + hr2[:, :D_OUT] + b2_ref[...]
    m = jnp.max(z, axis=1, keepdims=True)
    lse = jnp.log(jnp.sum(jnp.exp(z - m), axis=1, keepdims=True)) + m
    out_ref[...] = z - lse


def _make_sc_agg(d, k, nbuf, with_deg):
    assert EPT % k == 0
    nch = EPT // k
    nfull = (nch // nbuf) * nbuf
    rem = nch - nfull
    mesh = plsc.VectorSubcoreMesh(core_axis_name="c", subcore_axis_name="s",
                                  num_cores=NC, num_subcores=NS)
    out_type = [jax.ShapeDtypeStruct((NC, N, d), jnp.float32)]
    scratch = [
        pltpu.VMEM((nch, k), jnp.int32),
        pltpu.VMEM((nch, k), jnp.int32),
        [pltpu.VMEM((k, d), jnp.float32) for _ in range(nbuf)],
        pltpu.VMEM_SHARED((N, d), jnp.float32),
        [pltpu.SemaphoreType.DMA for _ in range(nbuf)],
        [pltpu.SemaphoreType.DMA for _ in range(nbuf)],
    ]
    if with_deg:
        out_type.append(jax.ShapeDtypeStruct((NC, N, DD), jnp.float32))
        scratch += [
            pltpu.VMEM((k, DD), jnp.float32),
            pltpu.VMEM_SHARED((N, DD), jnp.float32),
            [pltpu.SemaphoreType.DMA for _ in range(nbuf)],
        ]

    def body(refs):
        if with_deg:
            (table_hbm, ei_hbm, zeros_hbm, zerosd_hbm, ones_hbm, out_hbm,
             outd_hbm, src_v, dst_v, rows, acc, gsems, ssems, ones_v, dacc,
             dsems) = refs
        else:
            (table_hbm, ei_hbm, zeros_hbm, out_hbm,
             src_v, dst_v, rows, acc, gsems, ssems) = refs
        cid = lax.axis_index("c")
        sid = lax.axis_index("s")
        pltpu.sync_copy(ei_hbm.at[0, cid, sid], src_v)
        pltpu.sync_copy(ei_hbm.at[1, cid, sid], dst_v)
        if with_deg:
            pltpu.sync_copy(ones_hbm, ones_v)
        for b in range(nbuf):
            pltpu.async_copy(table_hbm.at[src_v.at[b]], rows[b], gsems[b])
        base = pl.multiple_of(sid * ROWS_PER_SUB, 8)
        pltpu.sync_copy(zeros_hbm.at[pl.ds(base, ROWS_PER_SUB)],
                        acc.at[pl.ds(base, ROWS_PER_SUB)])
        if with_deg:
            pltpu.sync_copy(zerosd_hbm.at[pl.ds(base, ROWS_PER_SUB)],
                            dacc.at[pl.ds(base, ROWS_PER_SUB)])

        @pl.when(sid == 0)
        def _():
            pltpu.sync_copy(zeros_hbm.at[pl.ds(ROWS_TAIL_BASE, ROWS_TAIL)],
                            acc.at[pl.ds(ROWS_TAIL_BASE, ROWS_TAIL)])
            if with_deg:
                pltpu.sync_copy(
                    zerosd_hbm.at[pl.ds(ROWS_TAIL_BASE, ROWS_TAIL)],
                    dacc.at[pl.ds(ROWS_TAIL_BASE, ROWS_TAIL)])

        plsc.subcore_barrier()

        def step(j, b):
            pltpu.make_async_copy(table_hbm.at[src_v.at[j]],
                                  rows[b], gsems[b]).wait()
            pltpu.async_copy(rows[b], acc.at[dst_v.at[j]], ssems[b],
                             add=True)
            if with_deg:
                pltpu.async_copy(ones_v, dacc.at[dst_v.at[j]], dsems[b],
                                 add=True)
            nxt = j + nbuf

            @pl.when(nxt < nch)
            def _():
                pltpu.make_async_copy(rows[b], acc.at[dst_v.at[j]],
                                      ssems[b]).wait()
                if with_deg:
                    pltpu.make_async_copy(ones_v, dacc.at[dst_v.at[j]],
                                          dsems[b]).wait()
                pltpu.async_copy(table_hbm.at[src_v.at[nxt]],
                                 rows[b], gsems[b])

        @pl.loop(0, nfull, step=nbuf)
        def _(g):
            for b in range(nbuf):
                step(g + b, b)

        for b in range(rem):
            step(nfull + b, b)

        for b in range(nbuf):
            pltpu.make_async_copy(rows[b], acc.at[dst_v.at[b]],
                                  ssems[b]).wait()
            if with_deg:
                pltpu.make_async_copy(ones_v, dacc.at[dst_v.at[b]],
                                      dsems[b]).wait()
        plsc.subcore_barrier()
        pltpu.sync_copy(acc.at[pl.ds(base, ROWS_PER_SUB)],
                        out_hbm.at[cid, pl.ds(base, ROWS_PER_SUB)])
        if with_deg:
            pltpu.sync_copy(dacc.at[pl.ds(base, ROWS_PER_SUB)],
                            outd_hbm.at[cid, pl.ds(base, ROWS_PER_SUB)])

        @pl.when(sid == 0)
        def _():
            pltpu.sync_copy(acc.at[pl.ds(ROWS_TAIL_BASE, ROWS_TAIL)],
                            out_hbm.at[cid, pl.ds(ROWS_TAIL_BASE, ROWS_TAIL)])
            if with_deg:
                pltpu.sync_copy(
                    dacc.at[pl.ds(ROWS_TAIL_BASE, ROWS_TAIL)],
                    outd_hbm.at[cid, pl.ds(ROWS_TAIL_BASE, ROWS_TAIL)])

    @functools.partial(
        pl.kernel,
        out_type=out_type if with_deg else out_type[0],
        mesh=mesh,
        scratch_types=scratch,
        compiler_params=pltpu.CompilerParams(use_tc_tiling_on_sc=False),
    )
    def sc_agg(*refs):
        body(refs)

    return sc_agg


_sc_agg1 = _make_sc_agg(D_IN, 40, 3, True)
_sc_agg2 = _make_sc_agg(D2, 80, 5, False)


def kernel(x, edge_index, W1_l, b1, W1_r, bn_gamma, bn_beta, bn_mean, bn_var,
           W2_l, b2, W2_r):
    ei1 = edge_index.reshape(2, NC, NS, EPT // 40, 40)
    ei2 = edge_index.reshape(2, NC, NS, EPT // 80, 80)
    zeros1 = jnp.zeros((N, D_IN), jnp.float32)
    zerosd = jnp.zeros((N, DD), jnp.float32)
    ones1 = jnp.ones((40, DD), jnp.float32)
    zeros2 = jnp.zeros((N, D2), jnp.float32)
    w2l_pad = jnp.pad(W2_l, ((0, 0), (0, D2 - D_OUT)))
    w2r_pad = jnp.pad(W2_r, ((0, 0), (0, D2 - D_OUT)))

    aggx, degp = _sc_agg1(x, ei1, zeros1, zerosd, ones1)

    param_spec = pl.BlockSpec((1, D_HID), lambda i: (0, 0))
    w_spec = pl.BlockSpec((D_HID, D_HID), lambda i: (0, 0))
    w2_spec = pl.BlockSpec((D_HID, D2), lambda i: (0, 0))
    t2p, hr2 = pl.pallas_call(
        _tc2_body,
        grid=(N // TC2_BLK,),
        in_specs=[
            pl.BlockSpec((NC, TC2_BLK, D_IN), lambda i: (0, i, 0)),
            pl.BlockSpec((NC, TC2_BLK, DD), lambda i: (0, i, 0)),
            pl.BlockSpec((TC2_BLK, D_IN), lambda i: (i, 0)),
            param_spec, param_spec, param_spec, param_spec, param_spec,
            w_spec, w_spec, w2_spec, w2_spec,
        ],
        out_specs=[
            pl.BlockSpec((TC2_BLK, D2), lambda i: (i, 0)),
            pl.BlockSpec((TC2_BLK, D2), lambda i: (i, 0)),
        ],
        out_shape=[jax.ShapeDtypeStruct((N, D2), jnp.float32),
                   jax.ShapeDtypeStruct((N, D2), jnp.float32)],
    )(aggx, degp, x, b1.reshape(1, -1), bn_gamma.reshape(1, -1),
      bn_beta.reshape(1, -1), bn_mean.reshape(1, -1), bn_var.reshape(1, -1),
      W1_l, W1_r, w2l_pad, w2r_pad)

    agg2 = _sc_agg2(t2p, ei2, zeros2)

    out = pl.pallas_call(
        _tc3_body,
        grid=(N // TC3_BLK,),
        in_specs=[
            pl.BlockSpec((NC, TC3_BLK, D2), lambda i: (0, i, 0)),
            pl.BlockSpec((TC3_BLK, D2), lambda i: (i, 0)),
            pl.BlockSpec((1, D_OUT), lambda i: (0, 0)),
        ],
        out_specs=pl.BlockSpec((TC3_BLK, D_OUT), lambda i: (i, 0)),
        out_shape=jax.ShapeDtypeStruct((N, D_OUT), jnp.float32),
    )(agg2, hr2, b2.reshape(1, -1))

    return out


# speedup vs baseline: 14.2718x; 1.0709x over previous
import functools

import jax
import jax.numpy as jnp
from jax import lax
from jax.experimental import pallas as pl
from jax.experimental.pallas import tpu as pltpu
from jax.experimental.pallas import tpu_sc as plsc

N = 10000
E = 320000
D_IN = 128
D_HID = 128
D_OUT = 40
DD = 16
D2 = 48
NC = 2
NS = 16
TILES = NC * NS
EPT = E // TILES
ROWS_PER_SUB = 624
ROWS_TAIL = N - NS * ROWS_PER_SUB
ROWS_TAIL_BASE = NS * ROWS_PER_SUB

_HIGH = jax.lax.Precision.DEFAULT


def _dot(a, b):
    return jax.lax.dot_general(a, b, (((1,), (0,)), ((), ())),
                               precision=_HIGH,
                               preferred_element_type=jnp.float32)


TC2_BLK = 1000


def _tc2_body(aggx_ref, degp_ref, x_ref, b1_ref, g_ref, be_ref, mu_ref,
              var_ref, w1l_ref, w1r_ref, w2l_ref, w2r_ref, t2p_ref, hr2_ref):
    axs = aggx_ref[0] + aggx_ref[1]
    deg = degp_ref[0, :, 0] + degp_ref[1, :, 0]
    inv = 1.0 / jnp.maximum(deg, 1.0)
    x = x_ref[...]
    pre = _dot(axs * inv[:, None], w1l_ref[...]) + _dot(x, w1r_ref[...]) \
        + b1_ref[...]
    h = (pre - mu_ref[...]) * jax.lax.rsqrt(var_ref[...] + 1e-5) \
        * g_ref[...] + be_ref[...]
    h = jnp.maximum(h, 0.0)
    t2p_ref[...] = _dot(h, w2l_ref[...])
    hr2 = _dot(h, w2r_ref[...])
    cols = jax.lax.broadcasted_iota(jnp.int32, (TC2_BLK, D2), 1)
    hr2_ref[...] = jnp.where(cols == D_OUT, deg[:, None], hr2)


TC3_BLK = 1000


def _tc3_body(agg2_ref, hr2_ref, b2_ref, out_ref):
    s = agg2_ref[0] + agg2_ref[1]
    hr2 = hr2_ref[...]
    deg = hr2[:, D_OUT]
    inv = 1.0 / jnp.maximum(deg, 1.0)
    z = s[:, :D_OUT] * inv[:, None] + hr2[:, :D_OUT] + b2_ref[...]
    m = jnp.max(z, axis=1, keepdims=True)
    lse = jnp.log(jnp.sum(jnp.exp(z - m), axis=1, keepdims=True)) + m
    out_ref[...] = z - lse


def _make_sc_agg(d, k, nbuf, with_deg):
    assert EPT % k == 0
    nch = EPT // k
    nfull = (nch // nbuf) * nbuf
    rem = nch - nfull
    mesh = plsc.VectorSubcoreMesh(core_axis_name="c", subcore_axis_name="s",
                                  num_cores=NC, num_subcores=NS)
    out_type = [jax.ShapeDtypeStruct((NC, N, d), jnp.float32)]
    scratch = [
        pltpu.VMEM((nch, k), jnp.int32),
        pltpu.VMEM((nch, k), jnp.int32),
        [pltpu.VMEM((k, d), jnp.float32) for _ in range(nbuf)],
        pltpu.VMEM_SHARED((N, d), jnp.float32),
        [pltpu.SemaphoreType.DMA for _ in range(nbuf)],
        [pltpu.SemaphoreType.DMA for _ in range(nbuf)],
    ]
    if with_deg:
        out_type.append(jax.ShapeDtypeStruct((NC, N, DD), jnp.float32))
        scratch += [
            pltpu.VMEM((k, DD), jnp.float32),
            pltpu.VMEM_SHARED((N, DD), jnp.float32),
            [pltpu.SemaphoreType.DMA for _ in range(nbuf)],
        ]

    def body(refs):
        if with_deg:
            (table_hbm, ei_hbm, zeros_hbm, zerosd_hbm, ones_hbm, out_hbm,
             outd_hbm, src_v, dst_v, rows, acc, gsems, ssems, ones_v, dacc,
             dsems) = refs
        else:
            (table_hbm, ei_hbm, zeros_hbm, out_hbm,
             src_v, dst_v, rows, acc, gsems, ssems) = refs
        cid = lax.axis_index("c")
        sid = lax.axis_index("s")
        pltpu.sync_copy(ei_hbm.at[0, cid, sid], src_v)
        pltpu.sync_copy(ei_hbm.at[1, cid, sid], dst_v)
        if with_deg:
            pltpu.sync_copy(ones_hbm, ones_v)
        for b in range(nbuf):
            pltpu.async_copy(table_hbm.at[src_v.at[b]], rows[b], gsems[b])
        base = pl.multiple_of(sid * ROWS_PER_SUB, 8)
        pltpu.sync_copy(zeros_hbm.at[pl.ds(base, ROWS_PER_SUB)],
                        acc.at[pl.ds(base, ROWS_PER_SUB)])
        if with_deg:
            pltpu.sync_copy(zerosd_hbm.at[pl.ds(base, ROWS_PER_SUB)],
                            dacc.at[pl.ds(base, ROWS_PER_SUB)])

        @pl.when(sid == 0)
        def _():
            pltpu.sync_copy(zeros_hbm.at[pl.ds(ROWS_TAIL_BASE, ROWS_TAIL)],
                            acc.at[pl.ds(ROWS_TAIL_BASE, ROWS_TAIL)])
            if with_deg:
                pltpu.sync_copy(
                    zerosd_hbm.at[pl.ds(ROWS_TAIL_BASE, ROWS_TAIL)],
                    dacc.at[pl.ds(ROWS_TAIL_BASE, ROWS_TAIL)])

        plsc.subcore_barrier()

        def step(j, b):
            pltpu.make_async_copy(table_hbm.at[src_v.at[j]],
                                  rows[b], gsems[b]).wait()
            pltpu.async_copy(rows[b], acc.at[dst_v.at[j]], ssems[b],
                             add=True)
            if with_deg:
                pltpu.async_copy(ones_v, dacc.at[dst_v.at[j]], dsems[b],
                                 add=True)
            nxt = j + nbuf

            @pl.when(nxt < nch)
            def _():
                pltpu.make_async_copy(rows[b], acc.at[dst_v.at[j]],
                                      ssems[b]).wait()
                if with_deg:
                    pltpu.make_async_copy(ones_v, dacc.at[dst_v.at[j]],
                                          dsems[b]).wait()
                pltpu.async_copy(table_hbm.at[src_v.at[nxt]],
                                 rows[b], gsems[b])

        @pl.loop(0, nfull, step=nbuf)
        def _(g):
            for b in range(nbuf):
                step(g + b, b)

        for b in range(rem):
            step(nfull + b, b)

        for b in range(nbuf):
            pltpu.make_async_copy(rows[b], acc.at[dst_v.at[b]],
                                  ssems[b]).wait()
            if with_deg:
                pltpu.make_async_copy(ones_v, dacc.at[dst_v.at[b]],
                                      dsems[b]).wait()
        plsc.subcore_barrier()
        pltpu.sync_copy(acc.at[pl.ds(base, ROWS_PER_SUB)],
                        out_hbm.at[cid, pl.ds(base, ROWS_PER_SUB)])
        if with_deg:
            pltpu.sync_copy(dacc.at[pl.ds(base, ROWS_PER_SUB)],
                            outd_hbm.at[cid, pl.ds(base, ROWS_PER_SUB)])

        @pl.when(sid == 0)
        def _():
            pltpu.sync_copy(acc.at[pl.ds(ROWS_TAIL_BASE, ROWS_TAIL)],
                            out_hbm.at[cid, pl.ds(ROWS_TAIL_BASE, ROWS_TAIL)])
            if with_deg:
                pltpu.sync_copy(
                    dacc.at[pl.ds(ROWS_TAIL_BASE, ROWS_TAIL)],
                    outd_hbm.at[cid, pl.ds(ROWS_TAIL_BASE, ROWS_TAIL)])

    @functools.partial(
        pl.kernel,
        out_type=out_type if with_deg else out_type[0],
        mesh=mesh,
        scratch_types=scratch,
        compiler_params=pltpu.CompilerParams(use_tc_tiling_on_sc=False),
    )
    def sc_agg(*refs):
        body(refs)

    return sc_agg


_sc_agg1 = _make_sc_agg(D_IN, 40, 3, True)
_sc_agg2 = _make_sc_agg(D2, 80, 5, False)


def kernel(x, edge_index, W1_l, b1, W1_r, bn_gamma, bn_beta, bn_mean, bn_var,
           W2_l, b2, W2_r):
    ei1 = edge_index.reshape(2, NC, NS, EPT // 40, 40)
    ei2 = edge_index.reshape(2, NC, NS, EPT // 80, 80)
    zeros1 = jnp.zeros((N, D_IN), jnp.float32)
    zerosd = jnp.zeros((N, DD), jnp.float32)
    ones1 = jnp.ones((40, DD), jnp.float32)
    zeros2 = jnp.zeros((N, D2), jnp.float32)
    w2l_pad = jnp.pad(W2_l, ((0, 0), (0, D2 - D_OUT)))
    w2r_pad = jnp.pad(W2_r, ((0, 0), (0, D2 - D_OUT)))

    aggx, degp = _sc_agg1(x, ei1, zeros1, zerosd, ones1)

    param_spec = pl.BlockSpec((1, D_HID), lambda i: (0, 0))
    w_spec = pl.BlockSpec((D_HID, D_HID), lambda i: (0, 0))
    w2_spec = pl.BlockSpec((D_HID, D2), lambda i: (0, 0))
    t2p, hr2 = pl.pallas_call(
        _tc2_body,
        grid=(N // TC2_BLK,),
        in_specs=[
            pl.BlockSpec((NC, TC2_BLK, D_IN), lambda i: (0, i, 0)),
            pl.BlockSpec((NC, TC2_BLK, DD), lambda i: (0, i, 0)),
            pl.BlockSpec((TC2_BLK, D_IN), lambda i: (i, 0)),
            param_spec, param_spec, param_spec, param_spec, param_spec,
            w_spec, w_spec, w2_spec, w2_spec,
        ],
        out_specs=[
            pl.BlockSpec((TC2_BLK, D2), lambda i: (i, 0)),
            pl.BlockSpec((TC2_BLK, D2), lambda i: (i, 0)),
        ],
        out_shape=[jax.ShapeDtypeStruct((N, D2), jnp.float32),
                   jax.ShapeDtypeStruct((N, D2), jnp.float32)],
    )(aggx, degp, x, b1.reshape(1, -1), bn_gamma.reshape(1, -1),
      bn_beta.reshape(1, -1), bn_mean.reshape(1, -1), bn_var.reshape(1, -1),
      W1_l, W1_r, w2l_pad, w2r_pad)

    agg2 = _sc_agg2(t2p, ei2, zeros2)

    out = pl.pallas_call(
        _tc3_body,
        grid=(N // TC3_BLK,),
        in_specs=[
            pl.BlockSpec((NC, TC3_BLK, D2), lambda i: (0, i, 0)),
            pl.BlockSpec((TC3_BLK, D2), lambda i: (i, 0)),
            pl.BlockSpec((1, D_OUT), lambda i: (0, 0)),
        ],
        out_specs=pl.BlockSpec((TC3_BLK, D_OUT), lambda i: (i, 0)),
        out_shape=jax.ShapeDtypeStruct((N, D_OUT), jnp.float32),
    )(agg2, hr2, b2.reshape(1, -1))

    return out


# speedup vs baseline: 14.4551x; 1.0128x over previous
import functools

import jax
import jax.numpy as jnp
from jax import lax
from jax.experimental import pallas as pl
from jax.experimental.pallas import tpu as pltpu
from jax.experimental.pallas import tpu_sc as plsc

N = 10000
E = 320000
D_IN = 128
D_HID = 128
D_OUT = 40
DD = 16
D2 = 48
NC = 2
NS = 16
TILES = NC * NS
EPT = E // TILES
ROWS_PER_SUB = 624
ROWS_TAIL = N - NS * ROWS_PER_SUB
ROWS_TAIL_BASE = NS * ROWS_PER_SUB

_HIGH = jax.lax.Precision.DEFAULT


def _dot(a, b):
    return jax.lax.dot_general(a, b, (((1,), (0,)), ((), ())),
                               precision=_HIGH,
                               preferred_element_type=jnp.float32)


TC2_BLK = 1000


def _tc2_body(aggx_ref, degp_ref, x_ref, b1_ref, g_ref, be_ref, mu_ref,
              var_ref, w1l_ref, w1r_ref, w2l_ref, w2r_ref, t2p_ref, hr2_ref):
    axs = aggx_ref[0] + aggx_ref[1]
    deg = degp_ref[0, :, 0] + degp_ref[1, :, 0]
    inv = 1.0 / jnp.maximum(deg, 1.0)
    x = x_ref[...]
    pre = _dot(axs * inv[:, None], w1l_ref[...]) + _dot(x, w1r_ref[...]) \
        + b1_ref[...]
    h = (pre - mu_ref[...]) * jax.lax.rsqrt(var_ref[...] + 1e-5) \
        * g_ref[...] + be_ref[...]
    h = jnp.maximum(h, 0.0)
    t2p_ref[...] = _dot(h, w2l_ref[...])
    hr2 = _dot(h, w2r_ref[...])
    cols = jax.lax.broadcasted_iota(jnp.int32, (TC2_BLK, D2), 1)
    hr2_ref[...] = jnp.where(cols == D_OUT, deg[:, None], hr2)


TC3_BLK = 2000


def _tc3_body(agg2_ref, hr2_ref, b2_ref, out_ref):
    s = agg2_ref[0] + agg2_ref[1]
    hr2 = hr2_ref[...]
    deg = hr2[:, D_OUT]
    inv = 1.0 / jnp.maximum(deg, 1.0)
    z = s[:, :D_OUT] * inv[:, None] + hr2[:, :D_OUT] + b2_ref[...]
    m = jnp.max(z, axis=1, keepdims=True)
    lse = jnp.log(jnp.sum(jnp.exp(z - m), axis=1, keepdims=True)) + m
    out_ref[...] = z - lse


def _make_sc_agg(d, k, nbuf, with_deg):
    assert EPT % k == 0
    nch = EPT // k
    nfull = (nch // nbuf) * nbuf
    rem = nch - nfull
    mesh = plsc.VectorSubcoreMesh(core_axis_name="c", subcore_axis_name="s",
                                  num_cores=NC, num_subcores=NS)
    out_type = [jax.ShapeDtypeStruct((NC, N, d), jnp.float32)]
    scratch = [
        pltpu.VMEM((nch, k), jnp.int32),
        pltpu.VMEM((nch, k), jnp.int32),
        [pltpu.VMEM((k, d), jnp.float32) for _ in range(nbuf)],
        pltpu.VMEM_SHARED((N, d), jnp.float32),
        [pltpu.SemaphoreType.DMA for _ in range(nbuf)],
        [pltpu.SemaphoreType.DMA for _ in range(nbuf)],
    ]
    if with_deg:
        out_type.append(jax.ShapeDtypeStruct((NC, N, DD), jnp.float32))
        scratch += [
            pltpu.VMEM((k, DD), jnp.float32),
            pltpu.VMEM_SHARED((N, DD), jnp.float32),
            [pltpu.SemaphoreType.DMA for _ in range(nbuf)],
        ]

    def body(refs):
        if with_deg:
            (table_hbm, ei_hbm, zeros_hbm, zerosd_hbm, ones_hbm, out_hbm,
             outd_hbm, src_v, dst_v, rows, acc, gsems, ssems, ones_v, dacc,
             dsems) = refs
        else:
            (table_hbm, ei_hbm, zeros_hbm, out_hbm,
             src_v, dst_v, rows, acc, gsems, ssems) = refs
        cid = lax.axis_index("c")
        sid = lax.axis_index("s")
        pltpu.sync_copy(ei_hbm.at[0, cid, sid], src_v)
        pltpu.sync_copy(ei_hbm.at[1, cid, sid], dst_v)
        if with_deg:
            pltpu.sync_copy(ones_hbm, ones_v)
        for b in range(nbuf):
            pltpu.async_copy(table_hbm.at[src_v.at[b]], rows[b], gsems[b])
        base = pl.multiple_of(sid * ROWS_PER_SUB, 8)
        pltpu.sync_copy(zeros_hbm.at[pl.ds(base, ROWS_PER_SUB)],
                        acc.at[pl.ds(base, ROWS_PER_SUB)])
        if with_deg:
            pltpu.sync_copy(zerosd_hbm.at[pl.ds(base, ROWS_PER_SUB)],
                            dacc.at[pl.ds(base, ROWS_PER_SUB)])

        @pl.when(sid == 0)
        def _():
            pltpu.sync_copy(zeros_hbm.at[pl.ds(ROWS_TAIL_BASE, ROWS_TAIL)],
                            acc.at[pl.ds(ROWS_TAIL_BASE, ROWS_TAIL)])
            if with_deg:
                pltpu.sync_copy(
                    zerosd_hbm.at[pl.ds(ROWS_TAIL_BASE, ROWS_TAIL)],
                    dacc.at[pl.ds(ROWS_TAIL_BASE, ROWS_TAIL)])

        plsc.subcore_barrier()

        def step(j, b):
            pltpu.make_async_copy(table_hbm.at[src_v.at[j]],
                                  rows[b], gsems[b]).wait()
            pltpu.async_copy(rows[b], acc.at[dst_v.at[j]], ssems[b],
                             add=True)
            if with_deg:
                pltpu.async_copy(ones_v, dacc.at[dst_v.at[j]], dsems[b],
                                 add=True)
            nxt = j + nbuf

            @pl.when(nxt < nch)
            def _():
                pltpu.make_async_copy(rows[b], acc.at[dst_v.at[j]],
                                      ssems[b]).wait()
                if with_deg:
                    pltpu.make_async_copy(ones_v, dacc.at[dst_v.at[j]],
                                          dsems[b]).wait()
                pltpu.async_copy(table_hbm.at[src_v.at[nxt]],
                                 rows[b], gsems[b])

        @pl.loop(0, nfull, step=nbuf)
        def _(g):
            for b in range(nbuf):
                step(g + b, b)

        for b in range(rem):
            step(nfull + b, b)

        for b in range(nbuf):
            pltpu.make_async_copy(rows[b], acc.at[dst_v.at[b]],
                                  ssems[b]).wait()
            if with_deg:
                pltpu.make_async_copy(ones_v, dacc.at[dst_v.at[b]],
                                      dsems[b]).wait()
        plsc.subcore_barrier()
        pltpu.sync_copy(acc.at[pl.ds(base, ROWS_PER_SUB)],
                        out_hbm.at[cid, pl.ds(base, ROWS_PER_SUB)])
        if with_deg:
            pltpu.sync_copy(dacc.at[pl.ds(base, ROWS_PER_SUB)],
                            outd_hbm.at[cid, pl.ds(base, ROWS_PER_SUB)])

        @pl.when(sid == 0)
        def _():
            pltpu.sync_copy(acc.at[pl.ds(ROWS_TAIL_BASE, ROWS_TAIL)],
                            out_hbm.at[cid, pl.ds(ROWS_TAIL_BASE, ROWS_TAIL)])
            if with_deg:
                pltpu.sync_copy(
                    dacc.at[pl.ds(ROWS_TAIL_BASE, ROWS_TAIL)],
                    outd_hbm.at[cid, pl.ds(ROWS_TAIL_BASE, ROWS_TAIL)])

    @functools.partial(
        pl.kernel,
        out_type=out_type if with_deg else out_type[0],
        mesh=mesh,
        scratch_types=scratch,
        compiler_params=pltpu.CompilerParams(use_tc_tiling_on_sc=False),
    )
    def sc_agg(*refs):
        body(refs)

    return sc_agg


_sc_agg1 = _make_sc_agg(D_IN, 40, 3, True)
_sc_agg2 = _make_sc_agg(D2, 100, 5, False)


def kernel(x, edge_index, W1_l, b1, W1_r, bn_gamma, bn_beta, bn_mean, bn_var,
           W2_l, b2, W2_r):
    ei1 = edge_index.reshape(2, NC, NS, EPT // 40, 40)
    ei2 = edge_index.reshape(2, NC, NS, EPT // 100, 100)
    zeros1 = jnp.zeros((N, D_IN), jnp.float32)
    zerosd = jnp.zeros((N, DD), jnp.float32)
    ones1 = jnp.ones((40, DD), jnp.float32)
    zeros2 = jnp.zeros((N, D2), jnp.float32)
    w2l_pad = jnp.pad(W2_l, ((0, 0), (0, D2 - D_OUT)))
    w2r_pad = jnp.pad(W2_r, ((0, 0), (0, D2 - D_OUT)))

    aggx, degp = _sc_agg1(x, ei1, zeros1, zerosd, ones1)

    param_spec = pl.BlockSpec((1, D_HID), lambda i: (0, 0))
    w_spec = pl.BlockSpec((D_HID, D_HID), lambda i: (0, 0))
    w2_spec = pl.BlockSpec((D_HID, D2), lambda i: (0, 0))
    t2p, hr2 = pl.pallas_call(
        _tc2_body,
        grid=(N // TC2_BLK,),
        in_specs=[
            pl.BlockSpec((NC, TC2_BLK, D_IN), lambda i: (0, i, 0)),
            pl.BlockSpec((NC, TC2_BLK, DD), lambda i: (0, i, 0)),
            pl.BlockSpec((TC2_BLK, D_IN), lambda i: (i, 0)),
            param_spec, param_spec, param_spec, param_spec, param_spec,
            w_spec, w_spec, w2_spec, w2_spec,
        ],
        out_specs=[
            pl.BlockSpec((TC2_BLK, D2), lambda i: (i, 0)),
            pl.BlockSpec((TC2_BLK, D2), lambda i: (i, 0)),
        ],
        out_shape=[jax.ShapeDtypeStruct((N, D2), jnp.float32),
                   jax.ShapeDtypeStruct((N, D2), jnp.float32)],
    )(aggx, degp, x, b1.reshape(1, -1), bn_gamma.reshape(1, -1),
      bn_beta.reshape(1, -1), bn_mean.reshape(1, -1), bn_var.reshape(1, -1),
      W1_l, W1_r, w2l_pad, w2r_pad)

    agg2 = _sc_agg2(t2p, ei2, zeros2)

    out = pl.pallas_call(
        _tc3_body,
        grid=(N // TC3_BLK,),
        in_specs=[
            pl.BlockSpec((NC, TC3_BLK, D2), lambda i: (0, i, 0)),
            pl.BlockSpec((TC3_BLK, D2), lambda i: (i, 0)),
            pl.BlockSpec((1, D_OUT), lambda i: (0, 0)),
        ],
        out_specs=pl.BlockSpec((TC3_BLK, D_OUT), lambda i: (i, 0)),
        out_shape=jax.ShapeDtypeStruct((N, D_OUT), jnp.float32),
    )(agg2, hr2, b2.reshape(1, -1))

    return out
